# baseline (speedup 1.0000x reference)
import functools

import jax
import jax.numpy as jnp
from jax import lax
from jax.experimental import pallas as pl
from jax.experimental.pallas import tpu as pltpu
from jax.experimental.pallas import tpu_sc as plsc

N = 10000
E = 320000
H = 128

NC = 2
NS = 16
NW = NC * NS
K = 80
CHUNKS = 128
EPAD = 327680

EK = 128
SCH = 1024
NSUBI = SCH // EK
SB0 = 4
SB1 = 16
NPAD = 10112
STRIPE = NPAD // NS

RB = 400
GRID = N // RB

_BN_S = 0.9999950000374997

_mesh = plsc.VectorSubcoreMesh(core_axis_name="c", subcore_axis_name="s")


@functools.partial(
    pl.kernel,
    out_type=jax.ShapeDtypeStruct((NC, NPAD, H), jnp.float32),
    mesh=_mesh,
    scratch_types=[
        pltpu.VMEM((CHUNKS, K), jnp.int32),
        pltpu.VMEM((K, H), jnp.float32),
        pltpu.VMEM_SHARED((NPAD, H), jnp.float32),
    ],
)
def _deg_kernel(dstr_hbm, zbig_hbm, e0_hbm, out_hbm, dst_v, ones_v, hist_sh):
    c = lax.axis_index("c")
    s = lax.axis_index("s")
    wid = c * NS + s
    pltpu.sync_copy(dstr_hbm.at[wid], dst_v)
    pltpu.sync_copy(e0_hbm, ones_v)
    pltpu.sync_copy(zbig_hbm.at[pl.ds(s * STRIPE, STRIPE)],
                    hist_sh.at[pl.ds(s * STRIPE, STRIPE)])
    plsc.subcore_barrier()

    @pl.loop(0, CHUNKS)
    def _(j):
        pltpu.sync_copy(ones_v, hist_sh.at[dst_v.at[j]], add=True)

    plsc.subcore_barrier()
    pltpu.sync_copy(hist_sh.at[pl.ds(s * STRIPE, STRIPE)],
                    out_hbm.at[c, pl.ds(s * STRIPE, STRIPE)])


@functools.partial(
    pl.kernel,
    out_type=jax.ShapeDtypeStruct((NC, NPAD, H), jnp.float32),
    mesh=_mesh,
    scratch_types=[
        pltpu.VMEM((2 * NSUBI, EK), jnp.int32),
        pltpu.VMEM((2 * NSUBI, EK), jnp.int32),
        pltpu.VMEM((EK, H), jnp.float32),
        pltpu.VMEM((EK, H), jnp.float32),
        pltpu.VMEM_SHARED((NPAD, H), jnp.float32),
        pltpu.SemaphoreType.DMA,
        pltpu.SemaphoreType.DMA,
        pltpu.SemaphoreType.DMA,
        pltpu.SemaphoreType.DMA,
    ],
)
def _edge_kernel(y_hbm, idx0_hbm, idx1_hbm, zbig_hbm, out_hbm,
                 idxa, idxb, buf_a, buf_b, acc_sh, sia, sib, sem_a, sem_b):
    c = lax.axis_index("c")
    s = lax.axis_index("s")
    pltpu.sync_copy(zbig_hbm.at[pl.ds(s * STRIPE, STRIPE)],
                    acc_sh.at[pl.ds(s * STRIPE, STRIPE)])
    plsc.subcore_barrier()

    def run_sb(y_ref, idxv):
        pltpu.make_async_copy(y_ref.at[idxv.at[NSUBI]], buf_a, sem_a).start()

        @pl.loop(0, NSUBI // 2)
        def _(jj):
            j0 = jj * 2
            j1 = j0 + 1
            pltpu.make_async_copy(y_ref.at[idxv.at[NSUBI + j0]], buf_a,
                                  sem_a).wait()
            pltpu.make_async_copy(y_ref.at[idxv.at[NSUBI + j1]], buf_b,
                                  sem_b).start()
            pltpu.sync_copy(buf_a, acc_sh.at[idxv.at[j0]], add=True)
            pltpu.make_async_copy(y_ref.at[idxv.at[NSUBI + j1]], buf_b,
                                  sem_b).wait()

            @pl.when(j0 + 2 < NSUBI)
            def _():
                pltpu.make_async_copy(y_ref.at[idxv.at[NSUBI + j0 + 2]],
                                      buf_a, sem_a).start()

            pltpu.sync_copy(buf_b, acc_sh.at[idxv.at[j1]], add=True)

    def run_all(y_ref, idx_ref, nsb):
        pltpu.make_async_copy(idx_ref.at[s, 0], idxa, sia).start()

        @pl.loop(0, nsb // 2)
        def _(ii):
            sb0 = ii * 2
            sb1 = sb0 + 1
            pltpu.make_async_copy(idx_ref.at[s, sb0], idxa, sia).wait()
            pltpu.make_async_copy(idx_ref.at[s, sb1], idxb, sib).start()
            run_sb(y_ref, idxa)
            pltpu.make_async_copy(idx_ref.at[s, sb1], idxb, sib).wait()

            @pl.when(sb0 + 2 < nsb)
            def _():
                pltpu.make_async_copy(idx_ref.at[s, sb0 + 2], idxa,
                                      sia).start()

            run_sb(y_ref, idxb)

    @pl.when(c == 0)
    def _():
        run_all(y_hbm, idx0_hbm, SB0)

    @pl.when(c == 1)
    def _():
        run_all(y_hbm, idx1_hbm, SB1)

    plsc.subcore_barrier()
    pltpu.sync_copy(acc_sh.at[pl.ds(s * STRIPE, STRIPE)],
                    out_hbm.at[c, pl.ds(s * STRIPE, STRIPE)])


def _mlp_body(x_ref, w0_ref, b0_ref, g0_ref, be0_ref,
              w1_ref, b1_ref, g1_ref, be1_ref, wg0_ref, degp_ref,
              xw0_ref, y0_ref, dinvb_ref):
    h = jnp.dot(x_ref[...], w0_ref[...], preferred_element_type=jnp.float32)
    h = jnp.maximum(h + b0_ref[...], 0.0) * g0_ref[...] + be0_ref[...]
    h = jnp.dot(h, w1_ref[...], preferred_element_type=jnp.float32)
    h = jnp.maximum(h + b1_ref[...], 0.0) * g1_ref[...] + be1_ref[...]
    xw0 = jnp.dot(h, wg0_ref[...], preferred_element_type=jnp.float32)
    cnt = degp_ref[0, :, 0:1] + degp_ref[1, :, 0:1]
    dinv = lax.rsqrt(cnt + 1.0)
    dinvb = jnp.broadcast_to(dinv, (RB, H))
    xw0_ref[...] = xw0
    y0_ref[...] = xw0 * dinvb
    dinvb_ref[...] = dinvb


def _mid_body(p_ref, xw_ref, dinvb_ref, wg1_ref, bg_ref, gg_ref, beg_ref,
              xw1_ref, y1_ref):
    dinvb = dinvb_ref[...]
    t = (p_ref[0] + p_ref[1]) * dinvb + xw_ref[...] * dinvb * dinvb + bg_ref[...]
    out0 = jnp.maximum(t, 0.0) * gg_ref[...] + beg_ref[...]
    xw1 = jnp.dot(out0, wg1_ref[...], preferred_element_type=jnp.float32)
    xw1_ref[...] = xw1
    y1_ref[...] = xw1 * dinvb


def _fin_body(q_ref, xw_ref, dinvb_ref, bg_ref, gg_ref, beg_ref, o_ref):
    dinvb = dinvb_ref[...]
    t = (q_ref[0] + q_ref[1]) * dinvb + xw_ref[...] * dinvb * dinvb + bg_ref[...]
    o_ref[...] = jnp.maximum(t, 0.0) * gg_ref[...] + beg_ref[...]


_row_spec = pl.BlockSpec((RB, H), lambda i: (i, 0))
_w_spec = pl.BlockSpec((H, H), lambda i: (0, 0))
_v_spec = pl.BlockSpec((1, H), lambda i: (0, 0))
_acc_spec = pl.BlockSpec((NC, RB, H), lambda i: (0, i, 0))

_mlp_call = pl.pallas_call(
    _mlp_body,
    grid=(GRID,),
    in_specs=[_row_spec, _w_spec, _v_spec, _v_spec, _v_spec,
              _w_spec, _v_spec, _v_spec, _v_spec, _w_spec,
              pl.BlockSpec((NC, RB, H), lambda i: (0, i, 0))],
    out_specs=[_row_spec, _row_spec, _row_spec],
    out_shape=[jax.ShapeDtypeStruct((N, H), jnp.float32)] * 3,
)

_mid_call = pl.pallas_call(
    _mid_body,
    grid=(GRID,),
    in_specs=[_acc_spec, _row_spec, _row_spec, _w_spec,
              _v_spec, _v_spec, _v_spec],
    out_specs=[_row_spec, _row_spec],
    out_shape=[jax.ShapeDtypeStruct((N, H), jnp.float32)] * 2,
)

_fin_call = pl.pallas_call(
    _fin_body,
    grid=(GRID,),
    in_specs=[_acc_spec, _row_spec, _row_spec, _v_spec, _v_spec, _v_spec],
    out_specs=_row_spec,
    out_shape=jax.ShapeDtypeStruct((N, H), jnp.float32),
)


def kernel(x, edge_index, w_mlp0, b_mlp0, gamma_mlp0, beta_mlp0,
           w_mlp1, b_mlp1, gamma_mlp1, beta_mlp1,
           w_gcn0, b_gcn0, gamma_gcn0, beta_gcn0,
           w_gcn1, b_gcn1, gamma_gcn1, beta_gcn1):
    src = edge_index[0]
    dst = edge_index[1]
    pad = EPAD - E
    srcp = jnp.concatenate([src, jnp.zeros((pad,), jnp.int32)])
    dstp = jnp.concatenate([dst, jnp.full((pad,), N, jnp.int32)])
    dstr = dstp.reshape(NW, CHUNKS, K)
    e_split = NS * SB0 * SCH
    def _mkidx(sv, dv, nsb):
        return jnp.concatenate([dv.reshape(NS, nsb, NSUBI, EK),
                                sv.reshape(NS, nsb, NSUBI, EK)], axis=2)
    idx0 = _mkidx(srcp[:e_split], dstp[:e_split], SB0)
    idx1 = _mkidx(srcp[e_split:], dstp[e_split:], SB1)
    zbig = jnp.zeros((NPAD, H), jnp.float32)
    e0rows = jnp.tile(
        (jnp.arange(H) == 0).astype(jnp.float32)[None, :], (K, 1))

    degp = _deg_kernel(dstr, zbig, e0rows)

    row = lambda v: v.reshape(1, H)
    xw0, y0, dinvb = _mlp_call(
        x, w_mlp0, row(b_mlp0), row(gamma_mlp0 * _BN_S), row(beta_mlp0),
        w_mlp1, row(b_mlp1), row(gamma_mlp1 * _BN_S), row(beta_mlp1),
        w_gcn0, degp)

    p = _edge_kernel(y0, idx0, idx1, zbig)
    xw1, y1 = _mid_call(p, xw0, dinvb, w_gcn1,
                        row(b_gcn0), row(gamma_gcn0 * _BN_S), row(beta_gcn0))

    q = _edge_kernel(y1, idx0, idx1, zbig)
    out = _fin_call(q, xw1, dinvb,
                    row(b_gcn1), row(gamma_gcn1 * _BN_S), row(beta_gcn1))
    return out


# speedup vs baseline: 1.0866x; 1.0866x over previous
import functools

import jax
import jax.numpy as jnp
from jax import lax
from jax.experimental import pallas as pl
from jax.experimental.pallas import tpu as pltpu
from jax.experimental.pallas import tpu_sc as plsc

N = 10000
E = 320000
H = 128

NC = 2
NS = 16
NW = NC * NS
K = 80
CHUNKS = 128
EPAD = 327680

EK = 128
SCH = 1024
NSUBI = SCH // EK
SB0 = 16
SB1 = 4
NPAD = 10112
STRIPE = NPAD // NS

RB = 400
GRID = N // RB

_BN_S = 0.9999950000374997

_mesh = plsc.VectorSubcoreMesh(core_axis_name="c", subcore_axis_name="s")


@functools.partial(
    pl.kernel,
    out_type=jax.ShapeDtypeStruct((NC, NPAD, H), jnp.float32),
    mesh=_mesh,
    scratch_types=[
        pltpu.VMEM((CHUNKS, K), jnp.int32),
        pltpu.VMEM((K, H), jnp.float32),
        pltpu.VMEM_SHARED((NPAD, H), jnp.float32),
    ],
)
def _deg_kernel(dstr_hbm, zbig_hbm, e0_hbm, out_hbm, dst_v, ones_v, hist_sh):
    c = lax.axis_index("c")
    s = lax.axis_index("s")
    wid = c * NS + s
    pltpu.sync_copy(dstr_hbm.at[wid], dst_v)
    pltpu.sync_copy(e0_hbm, ones_v)
    pltpu.sync_copy(zbig_hbm.at[pl.ds(s * STRIPE, STRIPE)],
                    hist_sh.at[pl.ds(s * STRIPE, STRIPE)])
    plsc.subcore_barrier()

    @pl.loop(0, CHUNKS)
    def _(j):
        pltpu.sync_copy(ones_v, hist_sh.at[dst_v.at[j]], add=True)

    plsc.subcore_barrier()
    pltpu.sync_copy(hist_sh.at[pl.ds(s * STRIPE, STRIPE)],
                    out_hbm.at[c, pl.ds(s * STRIPE, STRIPE)])


@functools.partial(
    pl.kernel,
    out_type=jax.ShapeDtypeStruct((NC, NPAD, H), jnp.float32),
    mesh=_mesh,
    scratch_types=[
        pltpu.VMEM((2 * NSUBI, EK), jnp.int32),
        pltpu.VMEM((2 * NSUBI, EK), jnp.int32),
        pltpu.VMEM((EK, H), jnp.float32),
        pltpu.VMEM((EK, H), jnp.float32),
        pltpu.VMEM_SHARED((NPAD, H), jnp.float32),
        pltpu.SemaphoreType.DMA,
        pltpu.SemaphoreType.DMA,
        pltpu.SemaphoreType.DMA,
        pltpu.SemaphoreType.DMA,
    ],
)
def _edge_kernel(y_hbm, idx0_hbm, idx1_hbm, zbig_hbm, out_hbm,
                 idxa, idxb, buf_a, buf_b, acc_sh, sia, sib, sem_a, sem_b):
    c = lax.axis_index("c")
    s = lax.axis_index("s")
    pltpu.sync_copy(zbig_hbm.at[pl.ds(s * STRIPE, STRIPE)],
                    acc_sh.at[pl.ds(s * STRIPE, STRIPE)])
    plsc.subcore_barrier()

    def run_sb(y_ref, idxv):
        pltpu.make_async_copy(y_ref.at[idxv.at[NSUBI]], buf_a, sem_a).start()

        @pl.loop(0, NSUBI // 2)
        def _(jj):
            j0 = jj * 2
            j1 = j0 + 1
            pltpu.make_async_copy(y_ref.at[idxv.at[NSUBI + j0]], buf_a,
                                  sem_a).wait()
            pltpu.make_async_copy(y_ref.at[idxv.at[NSUBI + j1]], buf_b,
                                  sem_b).start()
            pltpu.sync_copy(buf_a, acc_sh.at[idxv.at[j0]], add=True)
            pltpu.make_async_copy(y_ref.at[idxv.at[NSUBI + j1]], buf_b,
                                  sem_b).wait()

            @pl.when(j0 + 2 < NSUBI)
            def _():
                pltpu.make_async_copy(y_ref.at[idxv.at[NSUBI + j0 + 2]],
                                      buf_a, sem_a).start()

            pltpu.sync_copy(buf_b, acc_sh.at[idxv.at[j1]], add=True)

    def run_all(y_ref, idx_ref, nsb):
        pltpu.make_async_copy(idx_ref.at[s, 0], idxa, sia).start()

        @pl.loop(0, nsb // 2)
        def _(ii):
            sb0 = ii * 2
            sb1 = sb0 + 1
            pltpu.make_async_copy(idx_ref.at[s, sb0], idxa, sia).wait()
            pltpu.make_async_copy(idx_ref.at[s, sb1], idxb, sib).start()
            run_sb(y_ref, idxa)
            pltpu.make_async_copy(idx_ref.at[s, sb1], idxb, sib).wait()

            @pl.when(sb0 + 2 < nsb)
            def _():
                pltpu.make_async_copy(idx_ref.at[s, sb0 + 2], idxa,
                                      sia).start()

            run_sb(y_ref, idxb)

    @pl.when(c == 0)
    def _():
        run_all(y_hbm, idx0_hbm, SB0)

    @pl.when(c == 1)
    def _():
        run_all(y_hbm, idx1_hbm, SB1)

    plsc.subcore_barrier()
    pltpu.sync_copy(acc_sh.at[pl.ds(s * STRIPE, STRIPE)],
                    out_hbm.at[c, pl.ds(s * STRIPE, STRIPE)])


def _mlp_body(x_ref, w0_ref, b0_ref, g0_ref, be0_ref,
              w1_ref, b1_ref, g1_ref, be1_ref, wg0_ref, degp_ref,
              xw0_ref, y0_ref, dinvb_ref):
    h = jnp.dot(x_ref[...], w0_ref[...], preferred_element_type=jnp.float32)
    h = jnp.maximum(h + b0_ref[...], 0.0) * g0_ref[...] + be0_ref[...]
    h = jnp.dot(h, w1_ref[...], preferred_element_type=jnp.float32)
    h = jnp.maximum(h + b1_ref[...], 0.0) * g1_ref[...] + be1_ref[...]
    xw0 = jnp.dot(h, wg0_ref[...], preferred_element_type=jnp.float32)
    cnt = degp_ref[0, :, 0:1] + degp_ref[1, :, 0:1]
    dinv = lax.rsqrt(cnt + 1.0)
    dinvb = jnp.broadcast_to(dinv, (RB, H))
    xw0_ref[...] = xw0
    y0_ref[...] = xw0 * dinvb
    dinvb_ref[...] = dinvb


def _mid_body(p_ref, xw_ref, dinvb_ref, wg1_ref, bg_ref, gg_ref, beg_ref,
              xw1_ref, y1_ref):
    dinvb = dinvb_ref[...]
    t = (p_ref[0] + p_ref[1]) * dinvb + xw_ref[...] * dinvb * dinvb + bg_ref[...]
    out0 = jnp.maximum(t, 0.0) * gg_ref[...] + beg_ref[...]
    xw1 = jnp.dot(out0, wg1_ref[...], preferred_element_type=jnp.float32)
    xw1_ref[...] = xw1
    y1_ref[...] = xw1 * dinvb


def _fin_body(q_ref, xw_ref, dinvb_ref, bg_ref, gg_ref, beg_ref, o_ref):
    dinvb = dinvb_ref[...]
    t = (q_ref[0] + q_ref[1]) * dinvb + xw_ref[...] * dinvb * dinvb + bg_ref[...]
    o_ref[...] = jnp.maximum(t, 0.0) * gg_ref[...] + beg_ref[...]


_row_spec = pl.BlockSpec((RB, H), lambda i: (i, 0))
_w_spec = pl.BlockSpec((H, H), lambda i: (0, 0))
_v_spec = pl.BlockSpec((1, H), lambda i: (0, 0))
_acc_spec = pl.BlockSpec((NC, RB, H), lambda i: (0, i, 0))

_mlp_call = pl.pallas_call(
    _mlp_body,
    grid=(GRID,),
    in_specs=[_row_spec, _w_spec, _v_spec, _v_spec, _v_spec,
              _w_spec, _v_spec, _v_spec, _v_spec, _w_spec,
              pl.BlockSpec((NC, RB, H), lambda i: (0, i, 0))],
    out_specs=[_row_spec, _row_spec, _row_spec],
    out_shape=[jax.ShapeDtypeStruct((N, H), jnp.float32)] * 3,
)

_mid_call = pl.pallas_call(
    _mid_body,
    grid=(GRID,),
    in_specs=[_acc_spec, _row_spec, _row_spec, _w_spec,
              _v_spec, _v_spec, _v_spec],
    out_specs=[_row_spec, _row_spec],
    out_shape=[jax.ShapeDtypeStruct((N, H), jnp.float32)] * 2,
)

_fin_call = pl.pallas_call(
    _fin_body,
    grid=(GRID,),
    in_specs=[_acc_spec, _row_spec, _row_spec, _v_spec, _v_spec, _v_spec],
    out_specs=_row_spec,
    out_shape=jax.ShapeDtypeStruct((N, H), jnp.float32),
)


def kernel(x, edge_index, w_mlp0, b_mlp0, gamma_mlp0, beta_mlp0,
           w_mlp1, b_mlp1, gamma_mlp1, beta_mlp1,
           w_gcn0, b_gcn0, gamma_gcn0, beta_gcn0,
           w_gcn1, b_gcn1, gamma_gcn1, beta_gcn1):
    src = edge_index[0]
    dst = edge_index[1]
    pad = EPAD - E
    srcp = jnp.concatenate([src, jnp.zeros((pad,), jnp.int32)])
    dstp = jnp.concatenate([dst, jnp.full((pad,), N, jnp.int32)])
    dstr = dstp.reshape(NW, CHUNKS, K)
    e_split = NS * SB0 * SCH
    def _mkidx(sv, dv, nsb):
        return jnp.concatenate([dv.reshape(NS, nsb, NSUBI, EK),
                                sv.reshape(NS, nsb, NSUBI, EK)], axis=2)
    idx0 = _mkidx(srcp[:e_split], dstp[:e_split], SB0)
    idx1 = _mkidx(srcp[e_split:], dstp[e_split:], SB1)
    zbig = jnp.zeros((NPAD, H), jnp.float32)
    e0rows = jnp.tile(
        (jnp.arange(H) == 0).astype(jnp.float32)[None, :], (K, 1))

    degp = _deg_kernel(dstr, zbig, e0rows)

    row = lambda v: v.reshape(1, H)
    xw0, y0, dinvb = _mlp_call(
        x, w_mlp0, row(b_mlp0), row(gamma_mlp0 * _BN_S), row(beta_mlp0),
        w_mlp1, row(b_mlp1), row(gamma_mlp1 * _BN_S), row(beta_mlp1),
        w_gcn0, degp)

    p = _edge_kernel(y0, idx0, idx1, zbig)
    xw1, y1 = _mid_call(p, xw0, dinvb, w_gcn1,
                        row(b_gcn0), row(gamma_gcn0 * _BN_S), row(beta_gcn0))

    q = _edge_kernel(y1, idx0, idx1, zbig)
    out = _fin_call(q, xw1, dinvb,
                    row(b_gcn1), row(gamma_gcn1 * _BN_S), row(beta_gcn1))
    return out


# speedup vs baseline: 1.1337x; 1.0433x over previous
import functools

import jax
import jax.numpy as jnp
from jax import lax
from jax.experimental import pallas as pl
from jax.experimental.pallas import tpu as pltpu
from jax.experimental.pallas import tpu_sc as plsc

N = 10000
E = 320000
H = 128

NC = 2
NS = 16
NW = NC * NS
K = 80
CHUNKS = 128
EPAD = NW * CHUNKS * K
NPAD = 10112
STRIPE = NPAD // NS

RB = 400
GRID = N // RB

_BN_S = 0.9999950000374997

_mesh = plsc.VectorSubcoreMesh(core_axis_name="c", subcore_axis_name="s")


@functools.partial(
    pl.kernel,
    out_type=jax.ShapeDtypeStruct((NC, NPAD, H), jnp.float32),
    mesh=_mesh,
    scratch_types=[
        pltpu.VMEM((CHUNKS, K), jnp.int32),
        pltpu.VMEM((K, H), jnp.float32),
        pltpu.VMEM_SHARED((NPAD, H), jnp.float32),
    ],
)
def _deg_kernel(dstr_hbm, zbig_hbm, e0_hbm, out_hbm, dst_v, ones_v, hist_sh):
    c = lax.axis_index("c")
    s = lax.axis_index("s")
    wid = c * NS + s
    pltpu.sync_copy(dstr_hbm.at[wid], dst_v)
    pltpu.sync_copy(e0_hbm, ones_v)
    pltpu.sync_copy(zbig_hbm.at[pl.ds(s * STRIPE, STRIPE)],
                    hist_sh.at[pl.ds(s * STRIPE, STRIPE)])
    plsc.subcore_barrier()

    @pl.loop(0, CHUNKS)
    def _(j):
        pltpu.sync_copy(ones_v, hist_sh.at[dst_v.at[j]], add=True)

    plsc.subcore_barrier()
    pltpu.sync_copy(hist_sh.at[pl.ds(s * STRIPE, STRIPE)],
                    out_hbm.at[c, pl.ds(s * STRIPE, STRIPE)])


@functools.partial(
    pl.kernel,
    out_type=jax.ShapeDtypeStruct((NC, NPAD, H), jnp.float32),
    mesh=_mesh,
    scratch_types=[
        pltpu.VMEM((CHUNKS * K,), jnp.int32),
        pltpu.VMEM((CHUNKS, K), jnp.int32),
        pltpu.VMEM((K, H), jnp.float32),
        pltpu.VMEM((K, H), jnp.float32),
        pltpu.VMEM_SHARED((NPAD, H), jnp.float32),
        pltpu.SemaphoreType.DMA,
        pltpu.SemaphoreType.DMA,
    ],
)
def _edge_kernel(y_hbm, srcf_hbm, dstr_hbm, zbig_hbm, out_hbm,
                 src_v, dst_v, buf_a, buf_b, acc_sh, sem_a, sem_b):
    c = lax.axis_index("c")
    s = lax.axis_index("s")
    wid = c * NS + s
    pltpu.sync_copy(srcf_hbm.at[wid], src_v)
    pltpu.sync_copy(dstr_hbm.at[wid], dst_v)
    pltpu.sync_copy(zbig_hbm.at[pl.ds(s * STRIPE, STRIPE)],
                    acc_sh.at[pl.ds(s * STRIPE, STRIPE)])
    plsc.subcore_barrier()

    def _src(j):
        return src_v.at[pl.ds(j * K, K)]

    pltpu.make_async_copy(y_hbm.at[_src(0)], buf_a, sem_a).start()
    pltpu.make_async_copy(y_hbm.at[_src(1)], buf_b, sem_b).start()

    @pl.loop(0, CHUNKS // 2)
    def _(jj):
        j0 = jj * 2
        j1 = j0 + 1
        pltpu.make_async_copy(y_hbm.at[_src(j0)], buf_a, sem_a).wait()
        pltpu.sync_copy(buf_a, acc_sh.at[dst_v.at[j0]], add=True)

        @pl.when(j0 + 2 < CHUNKS)
        def _():
            pltpu.make_async_copy(y_hbm.at[_src(j0 + 2)], buf_a, sem_a).start()

        pltpu.make_async_copy(y_hbm.at[_src(j1)], buf_b, sem_b).wait()
        pltpu.sync_copy(buf_b, acc_sh.at[dst_v.at[j1]], add=True)

        @pl.when(j1 + 2 < CHUNKS)
        def _():
            pltpu.make_async_copy(y_hbm.at[_src(j1 + 2)], buf_b, sem_b).start()

    plsc.subcore_barrier()
    pltpu.sync_copy(acc_sh.at[pl.ds(s * STRIPE, STRIPE)],
                    out_hbm.at[c, pl.ds(s * STRIPE, STRIPE)])


def _mlp_body(x_ref, w0_ref, b0_ref, g0_ref, be0_ref,
              w1_ref, b1_ref, g1_ref, be1_ref, wg0_ref, degp_ref,
              xw0_ref, y0_ref, dinvb_ref):
    h = jnp.dot(x_ref[...], w0_ref[...], preferred_element_type=jnp.float32)
    h = jnp.maximum(h + b0_ref[...], 0.0) * g0_ref[...] + be0_ref[...]
    h = jnp.dot(h, w1_ref[...], preferred_element_type=jnp.float32)
    h = jnp.maximum(h + b1_ref[...], 0.0) * g1_ref[...] + be1_ref[...]
    xw0 = jnp.dot(h, wg0_ref[...], preferred_element_type=jnp.float32)
    cnt = degp_ref[0, :, 0:1] + degp_ref[1, :, 0:1]
    dinv = lax.rsqrt(cnt + 1.0)
    dinvb = jnp.broadcast_to(dinv, (RB, H))
    xw0_ref[...] = xw0
    y0_ref[...] = xw0 * dinvb
    dinvb_ref[...] = dinvb


def _mid_body(p_ref, xw_ref, dinvb_ref, wg1_ref, bg_ref, gg_ref, beg_ref,
              xw1_ref, y1_ref):
    dinvb = dinvb_ref[...]
    t = (p_ref[0] + p_ref[1]) * dinvb + xw_ref[...] * dinvb * dinvb + bg_ref[...]
    out0 = jnp.maximum(t, 0.0) * gg_ref[...] + beg_ref[...]
    xw1 = jnp.dot(out0, wg1_ref[...], preferred_element_type=jnp.float32)
    xw1_ref[...] = xw1
    y1_ref[...] = xw1 * dinvb


def _fin_body(q_ref, xw_ref, dinvb_ref, bg_ref, gg_ref, beg_ref, o_ref):
    dinvb = dinvb_ref[...]
    t = (q_ref[0] + q_ref[1]) * dinvb + xw_ref[...] * dinvb * dinvb + bg_ref[...]
    o_ref[...] = jnp.maximum(t, 0.0) * gg_ref[...] + beg_ref[...]


_row_spec = pl.BlockSpec((RB, H), lambda i: (i, 0))
_w_spec = pl.BlockSpec((H, H), lambda i: (0, 0))
_v_spec = pl.BlockSpec((1, H), lambda i: (0, 0))
_acc_spec = pl.BlockSpec((NC, RB, H), lambda i: (0, i, 0))

_mlp_call = pl.pallas_call(
    _mlp_body,
    grid=(GRID,),
    in_specs=[_row_spec, _w_spec, _v_spec, _v_spec, _v_spec,
              _w_spec, _v_spec, _v_spec, _v_spec, _w_spec,
              pl.BlockSpec((NC, RB, H), lambda i: (0, i, 0))],
    out_specs=[_row_spec, _row_spec, _row_spec],
    out_shape=[jax.ShapeDtypeStruct((N, H), jnp.float32)] * 3,
)

_mid_call = pl.pallas_call(
    _mid_body,
    grid=(GRID,),
    in_specs=[_acc_spec, _row_spec, _row_spec, _w_spec,
              _v_spec, _v_spec, _v_spec],
    out_specs=[_row_spec, _row_spec],
    out_shape=[jax.ShapeDtypeStruct((N, H), jnp.float32)] * 2,
)

_fin_call = pl.pallas_call(
    _fin_body,
    grid=(GRID,),
    in_specs=[_acc_spec, _row_spec, _row_spec, _v_spec, _v_spec, _v_spec],
    out_specs=_row_spec,
    out_shape=jax.ShapeDtypeStruct((N, H), jnp.float32),
)


def kernel(x, edge_index, w_mlp0, b_mlp0, gamma_mlp0, beta_mlp0,
           w_mlp1, b_mlp1, gamma_mlp1, beta_mlp1,
           w_gcn0, b_gcn0, gamma_gcn0, beta_gcn0,
           w_gcn1, b_gcn1, gamma_gcn1, beta_gcn1):
    src = edge_index[0]
    dst = edge_index[1]
    pad = EPAD - E
    srcp = jnp.concatenate([src, jnp.zeros((pad,), jnp.int32)])
    dstp = jnp.concatenate([dst, jnp.full((pad,), N, jnp.int32)])
    srcf = srcp.reshape(NW, CHUNKS * K)
    dstr = dstp.reshape(NW, CHUNKS, K)
    zbig = jnp.zeros((NPAD, H), jnp.float32)
    e0rows = jnp.tile(
        (jnp.arange(H) == 0).astype(jnp.float32)[None, :], (K, 1))

    degp = _deg_kernel(dstr, zbig, e0rows)

    row = lambda v: v.reshape(1, H)
    xw0, y0, dinvb = _mlp_call(
        x, w_mlp0, row(b_mlp0), row(gamma_mlp0 * _BN_S), row(beta_mlp0),
        w_mlp1, row(b_mlp1), row(gamma_mlp1 * _BN_S), row(beta_mlp1),
        w_gcn0, degp)

    p = _edge_kernel(y0, srcf, dstr, zbig)
    xw1, y1 = _mid_call(p, xw0, dinvb, w_gcn1,
                        row(b_gcn0), row(gamma_gcn0 * _BN_S), row(beta_gcn0))

    q = _edge_kernel(y1, srcf, dstr, zbig)
    out = _fin_call(q, xw1, dinvb,
                    row(b_gcn1), row(gamma_gcn1 * _BN_S), row(beta_gcn1))
    return out
